# baseline (speedup 1.0000x reference)
import functools

import jax
import jax.numpy as jnp
from jax import lax
from jax.experimental import pallas as pl
from jax.experimental.pallas import tpu as pltpu
from jax.experimental.pallas import tpu_sc as plsc

_INFO = plsc.get_sparse_core_info()
_NC, _NS, _L = _INFO.num_cores, _INFO.num_subcores, _INFO.num_lanes
_NW = _NC * _NS

_B = 16384
_D = 64
_BPW = _B // _NW
_GCHUNK = 128
_NG = _BPW // _GCHUNK


def _gmf_body(uidx_hbm, iidx_hbm, utab_hbm, itab_hbm, w_hbm, b_hbm,
              out_hbm,
              uidx_v, iidx_v, urows, irows, w_v, b_v, accbuf, outbuf, sem):
    wid = lax.axis_index("s") * _NC + lax.axis_index("c")
    base = wid * _BPW

    pltpu.sync_copy(uidx_hbm.at[wid], uidx_v)
    pltpu.sync_copy(iidx_hbm.at[wid], iidx_v)
    pltpu.sync_copy(w_hbm, w_v)
    pltpu.sync_copy(b_hbm, b_v)

    copies = []
    for j in range(_NG):
        copies.append(pltpu.async_copy(
            utab_hbm.at[uidx_v.at[j]], urows.at[pl.ds(j * _GCHUNK, _GCHUNK)],
            sem))
        copies.append(pltpu.async_copy(
            itab_hbm.at[iidx_v.at[j]], irows.at[pl.ds(j * _GCHUNK, _GCHUNK)],
            sem))
    for c in copies:
        c.wait()

    w0 = w_v[pl.ds(0, _L)]
    w1 = w_v[pl.ds(_L, _L)]
    w2 = w_v[pl.ds(2 * _L, _L)]
    w3 = w_v[pl.ds(3 * _L, _L)]
    bias = b_v[...]

    def row_body(r, carry):
        acc = (urows[r, pl.ds(0, _L)] * irows[r, pl.ds(0, _L)] * w0
               + urows[r, pl.ds(_L, _L)] * irows[r, pl.ds(_L, _L)] * w1
               + urows[r, pl.ds(2 * _L, _L)] * irows[r, pl.ds(2 * _L, _L)] * w2
               + urows[r, pl.ds(3 * _L, _L)] * irows[r, pl.ds(3 * _L, _L)] * w3)
        accbuf[pl.ds(r * 17, _L)] = acc
        return carry

    lax.fori_loop(0, _BPW, row_body, 0)

    iota16 = lax.iota(jnp.int32, _L)

    def grp_body(g, carry):
        flat = iota16 * 17 + g * (_L * 17)
        acc = bias
        for l in range(_L):
            acc = acc + plsc.load_gather(accbuf, [flat + l])
        outbuf[pl.ds(g * _L, _L)] = 1.0 / (1.0 + jnp.exp(-acc))
        return carry

    lax.fori_loop(0, _BPW // _L, grp_body, 0)

    pltpu.sync_copy(outbuf, out_hbm.at[pl.ds(base, _BPW)])


@functools.partial(jax.jit, static_argnames=())
def _gmf_call(uidx, iidx, utab, itab, w_flat, b_vec):
    mesh = plsc.VectorSubcoreMesh(core_axis_name="c", subcore_axis_name="s")
    return pl.kernel(
        _gmf_body,
        mesh=mesh,
        out_type=jax.ShapeDtypeStruct((_B,), jnp.float32),
        scratch_types=[
            pltpu.VMEM((_NG, _GCHUNK), jnp.int32),
            pltpu.VMEM((_NG, _GCHUNK), jnp.int32),
            pltpu.VMEM((_BPW, _D), jnp.float32),
            pltpu.VMEM((_BPW, _D), jnp.float32),
            pltpu.VMEM((_D,), jnp.float32),
            pltpu.VMEM((_L,), jnp.float32),
            pltpu.VMEM((_BPW * 17,), jnp.float32),
            pltpu.VMEM((_BPW,), jnp.float32),
            pltpu.SemaphoreType.DMA,
        ],
        compiler_params=pltpu.CompilerParams(
            needs_layout_passes=False, use_tc_tiling_on_sc=False),
    )(uidx, iidx, utab, itab, w_flat, b_vec)


def kernel(user_indices, item_indices, user_table, item_table, affine_w,
           affine_b):
    uidx = user_indices.astype(jnp.int32).reshape(_NW, _NG, _GCHUNK)
    iidx = item_indices.astype(jnp.int32).reshape(_NW, _NG, _GCHUNK)
    w_flat = affine_w.reshape(_D)
    b_vec = jnp.broadcast_to(affine_b.reshape(()), (_L,))
    out = _gmf_call(uidx, iidx, user_table, item_table, w_flat, b_vec)
    return out.reshape(_B, 1)
